# Initial kernel scaffold; baseline (speedup 1.0000x reference)
import jax
import jax.numpy as jnp
from jax.experimental import pallas as pl


def kernel(proposals, objectness, num_anchors_per_level):
    raise NotImplementedError("write your pallas kernel here")


# speedup vs baseline: 11.6364x; 11.6364x over previous
import jax
import jax.numpy as jnp
from jax import lax
from jax.experimental import pallas as pl
from jax.experimental.pallas import tpu as pltpu

_NAPL = (49152, 12288, 3072, 768, 192)
_PRE_NMS_TOP_N = 1000
_POST_NMS_TOP_N = 1000
_NMS_THRESH = 0.7
_SCORE_THRESH = 0.0
_MIN_SIZE = 0.001
_IMG_H = 512.0
_IMG_W = 512.0
_P = 1024


def _nms_kernel(sc_ref, x1_ref, y1_ref, x2_ref, y2_ref,
                ks_ref, ox1_ref, oy1_ref, ox2_ref, oy2_ref, iou_ref):
    logit = sc_ref[0, 0, 0, :]
    x1 = jnp.clip(x1_ref[0, 0, 0, :], 0.0, _IMG_W)
    y1 = jnp.clip(y1_ref[0, 0, 0, :], 0.0, _IMG_H)
    x2 = jnp.clip(x2_ref[0, 0, 0, :], 0.0, _IMG_W)
    y2 = jnp.clip(y2_ref[0, 0, 0, :], 0.0, _IMG_H)
    sig = jax.nn.sigmoid(logit)
    ws = x2 - x1
    hs = y2 - y1
    valid = (ws >= _MIN_SIZE) & (hs >= _MIN_SIZE) & (sig >= _SCORE_THRESH)
    zero = jnp.zeros_like(x1)
    x1 = jnp.where(valid, x1, zero)
    y1 = jnp.where(valid, y1, zero)
    x2 = jnp.where(valid, x2, zero)
    y2 = jnp.where(valid, y2, zero)
    area = (x2 - x1) * (y2 - y1)
    xx1 = jnp.maximum(x1[:, None], x1[None, :])
    yy1 = jnp.maximum(y1[:, None], y1[None, :])
    xx2 = jnp.minimum(x2[:, None], x2[None, :])
    yy2 = jnp.minimum(y2[:, None], y2[None, :])
    inter = jnp.clip(xx2 - xx1, 0.0) * jnp.clip(yy2 - yy1, 0.0)
    union = area[:, None] + area[None, :] - inter
    iou_ref[:, :] = jnp.where(
        union > 0.0, inter / jnp.where(union > 0.0, union, 1.0), 0.0)

    idx = lax.broadcasted_iota(jnp.int32, (1, _P), 1)

    def body(i, supp):
        row = iou_ref[pl.ds(i, 1), :]
        sup_i = jnp.max(jnp.where(idx == i, supp, 0.0))
        alive = 1.0 - sup_i
        cand = jnp.where((row > _NMS_THRESH) & (idx > i), alive, 0.0)
        return jnp.maximum(supp, cand)

    supp0 = jnp.zeros((1, _P), dtype=jnp.float32)
    supp = lax.fori_loop(0, _P, body, supp0)
    keep = valid & (supp[0] == 0.0)
    ks_ref[0, 0, 0, :] = jnp.where(keep, sig, -jnp.inf)
    ox1_ref[0, 0, 0, :] = x1
    oy1_ref[0, 0, 0, :] = y1
    ox2_ref[0, 0, 0, :] = x2
    oy2_ref[0, 0, 0, :] = y2


def _run_nms(scores_p, x1, y1, x2, y2):
    ni, nl, _ = scores_p.shape
    r = lambda a: a.reshape(ni, nl, 1, _P)
    spec = pl.BlockSpec((1, 1, 1, _P), lambda i, l: (i, l, 0, 0))
    shp = jax.ShapeDtypeStruct((ni, nl, 1, _P), jnp.float32)
    outs = pl.pallas_call(
        _nms_kernel,
        grid=(ni, nl),
        in_specs=[spec] * 5,
        out_specs=[spec] * 5,
        out_shape=[shp] * 5,
        scratch_shapes=[pltpu.VMEM((_P, _P), jnp.float32)],
    )(r(scores_p), r(x1), r(y1), r(x2), r(y2))
    return [o.reshape(ni, nl, _P) for o in outs]


def kernel(proposals, objectness, num_anchors_per_level):
    ni = objectness.shape[0]
    obj = objectness.reshape(ni, -1).astype(jnp.float32)
    props = proposals.astype(jnp.float32)
    off = 0
    sc, bx = [], []
    for n in _NAPL:
        k = min(_PRE_NMS_TOP_N, n)
        v, idx = lax.top_k(obj[:, off:off + n], k)
        b = jnp.take_along_axis(props[:, off:off + n, :], idx[:, :, None],
                                axis=1)
        pad = _P - k
        v = jnp.pad(v, ((0, 0), (0, pad)), constant_values=-jnp.inf)
        b = jnp.pad(b, ((0, 0), (0, pad), (0, 0)))
        sc.append(v)
        bx.append(b)
        off += n
    scores_p = jnp.stack(sc, 1)
    boxes_p = jnp.stack(bx, 1)
    ks, ox1, oy1, ox2, oy2 = _run_nms(
        scores_p, boxes_p[..., 0], boxes_p[..., 1],
        boxes_p[..., 2], boxes_p[..., 3])
    flat_ks = ks.reshape(ni, -1)
    topv, topi = lax.top_k(flat_ks, _POST_NMS_TOP_N)
    fb = jnp.stack([ox1, oy1, ox2, oy2], -1).reshape(ni, -1, 4)
    selb = jnp.take_along_axis(fb, topi[:, :, None], axis=1)
    finv = topv > -jnp.inf
    boxes_out = jnp.where(finv[:, :, None], selb, 0.0)
    scores_out = jnp.where(finv, topv, 0.0)
    return boxes_out, scores_out


# speedup vs baseline: 14.2464x; 1.2243x over previous
import jax
import jax.numpy as jnp
from jax import lax
from jax.experimental import pallas as pl
from jax.experimental.pallas import tpu as pltpu

_NAPL = (49152, 12288, 3072, 768, 192)
_PRE_NMS_TOP_N = 1000
_POST_NMS_TOP_N = 1000
_NMS_THRESH = 0.7
_SCORE_THRESH = 0.0
_MIN_SIZE = 0.001
_IMG_H = 512.0
_IMG_W = 512.0
_P = 1024


def _nms_kernel(sc_ref, x1_ref, y1_ref, x2_ref, y2_ref,
                ks_ref, ox1_ref, oy1_ref, ox2_ref, oy2_ref, iou_ref):
    logit = sc_ref[0, 0, 0, :]
    x1 = jnp.clip(x1_ref[0, 0, 0, :], 0.0, _IMG_W)
    y1 = jnp.clip(y1_ref[0, 0, 0, :], 0.0, _IMG_H)
    x2 = jnp.clip(x2_ref[0, 0, 0, :], 0.0, _IMG_W)
    y2 = jnp.clip(y2_ref[0, 0, 0, :], 0.0, _IMG_H)
    sig = jax.nn.sigmoid(logit)
    ws = x2 - x1
    hs = y2 - y1
    valid = (ws >= _MIN_SIZE) & (hs >= _MIN_SIZE) & (sig >= _SCORE_THRESH)
    zero = jnp.zeros_like(x1)
    x1 = jnp.where(valid, x1, zero)
    y1 = jnp.where(valid, y1, zero)
    x2 = jnp.where(valid, x2, zero)
    y2 = jnp.where(valid, y2, zero)
    area = (x2 - x1) * (y2 - y1)
    xx1 = jnp.maximum(x1[:, None], x1[None, :])
    yy1 = jnp.maximum(y1[:, None], y1[None, :])
    xx2 = jnp.minimum(x2[:, None], x2[None, :])
    yy2 = jnp.minimum(y2[:, None], y2[None, :])
    inter = jnp.clip(xx2 - xx1, 0.0) * jnp.clip(yy2 - yy1, 0.0)
    union = area[:, None] + area[None, :] - inter
    iou_ref[:, :] = jnp.where(
        union > 0.0, inter / jnp.where(union > 0.0, union, 1.0), 0.0)

    idx = lax.broadcasted_iota(jnp.int32, (1, _P), 1)

    def body(i, supp):
        row = iou_ref[pl.ds(i, 1), :]
        sup_i = jnp.max(jnp.where(idx == i, supp, 0.0))
        alive = 1.0 - sup_i
        cand = jnp.where((row > _NMS_THRESH) & (idx > i), alive, 0.0)
        return jnp.maximum(supp, cand)

    lvl = pl.program_id(1)
    trip = jnp.where(lvl == 3, 768, jnp.where(lvl == 4, 192, 1000))
    supp0 = jnp.zeros((1, _P), dtype=jnp.float32)
    supp = lax.fori_loop(0, trip, body, supp0)
    keep = valid & (supp[0] == 0.0)
    ks_ref[0, 0, 0, :] = jnp.where(keep, sig, -jnp.inf)
    ox1_ref[0, 0, 0, :] = x1
    oy1_ref[0, 0, 0, :] = y1
    ox2_ref[0, 0, 0, :] = x2
    oy2_ref[0, 0, 0, :] = y2


def _run_nms(scores_p, x1, y1, x2, y2):
    ni, nl, _ = scores_p.shape
    r = lambda a: a.reshape(ni, nl, 1, _P)
    spec = pl.BlockSpec((1, 1, 1, _P), lambda i, l: (i, l, 0, 0))
    shp = jax.ShapeDtypeStruct((ni, nl, 1, _P), jnp.float32)
    outs = pl.pallas_call(
        _nms_kernel,
        grid=(ni, nl),
        in_specs=[spec] * 5,
        out_specs=[spec] * 5,
        out_shape=[shp] * 5,
        scratch_shapes=[pltpu.VMEM((_P, _P), jnp.float32)],
    )(r(scores_p), r(x1), r(y1), r(x2), r(y2))
    return [o.reshape(ni, nl, _P) for o in outs]


def kernel(proposals, objectness, num_anchors_per_level):
    ni = objectness.shape[0]
    obj = objectness.reshape(ni, -1).astype(jnp.float32)
    props = proposals.astype(jnp.float32)
    off = 0
    sc, bx = [], []
    for n in _NAPL:
        k = min(_PRE_NMS_TOP_N, n)
        v, idx = lax.top_k(obj[:, off:off + n], k)
        b = jnp.take_along_axis(props[:, off:off + n, :], idx[:, :, None],
                                axis=1)
        pad = _P - k
        v = jnp.pad(v, ((0, 0), (0, pad)), constant_values=-jnp.inf)
        b = jnp.pad(b, ((0, 0), (0, pad), (0, 0)))
        sc.append(v)
        bx.append(b)
        off += n
    scores_p = jnp.stack(sc, 1)
    boxes_p = jnp.stack(bx, 1)
    ks, ox1, oy1, ox2, oy2 = _run_nms(
        scores_p, boxes_p[..., 0], boxes_p[..., 1],
        boxes_p[..., 2], boxes_p[..., 3])
    flat_ks = ks.reshape(ni, -1)
    topv, topi = lax.top_k(flat_ks, _POST_NMS_TOP_N)
    fb = jnp.stack([ox1, oy1, ox2, oy2], -1).reshape(ni, -1, 4)
    selb = jnp.take_along_axis(fb, topi[:, :, None], axis=1)
    finv = topv > -jnp.inf
    boxes_out = jnp.where(finv[:, :, None], selb, 0.0)
    scores_out = jnp.where(finv, topv, 0.0)
    return boxes_out, scores_out


# speedup vs baseline: 14.2948x; 1.0034x over previous
import jax
import jax.numpy as jnp
from jax import lax
from jax.experimental import pallas as pl
from jax.experimental.pallas import tpu as pltpu

_NAPL = (49152, 12288, 3072, 768, 192)
_PRE_NMS_TOP_N = 1000
_POST_NMS_TOP_N = 1000
_NMS_THRESH = 0.7
_SCORE_THRESH = 0.0
_MIN_SIZE = 0.001
_IMG_H = 512.0
_IMG_W = 512.0
_P = 1024


def _nms_kernel(sc_ref, x1_ref, y1_ref, x2_ref, y2_ref,
                ks_ref, ox1_ref, oy1_ref, ox2_ref, oy2_ref, iou_ref):
    logit = sc_ref[0, 0, 0, :]
    x1 = jnp.clip(x1_ref[0, 0, 0, :], 0.0, _IMG_W)
    y1 = jnp.clip(y1_ref[0, 0, 0, :], 0.0, _IMG_H)
    x2 = jnp.clip(x2_ref[0, 0, 0, :], 0.0, _IMG_W)
    y2 = jnp.clip(y2_ref[0, 0, 0, :], 0.0, _IMG_H)
    sig = jax.nn.sigmoid(logit)
    ws = x2 - x1
    hs = y2 - y1
    valid = (ws >= _MIN_SIZE) & (hs >= _MIN_SIZE) & (sig >= _SCORE_THRESH)
    zero = jnp.zeros_like(x1)
    x1 = jnp.where(valid, x1, zero)
    y1 = jnp.where(valid, y1, zero)
    x2 = jnp.where(valid, x2, zero)
    y2 = jnp.where(valid, y2, zero)
    area = (x2 - x1) * (y2 - y1)
    xx1 = jnp.maximum(x1[:, None], x1[None, :])
    yy1 = jnp.maximum(y1[:, None], y1[None, :])
    xx2 = jnp.minimum(x2[:, None], x2[None, :])
    yy2 = jnp.minimum(y2[:, None], y2[None, :])
    inter = jnp.clip(xx2 - xx1, 0.0) * jnp.clip(yy2 - yy1, 0.0)
    union = area[:, None] + area[None, :] - inter
    iou_ref[:, :] = jnp.where(
        union > 0.0, inter / jnp.where(union > 0.0, union, 1.0), 0.0)

    idx = lax.broadcasted_iota(jnp.int32, (1, _P), 1)

    def body(i, supp):
        row = iou_ref[pl.ds(i, 1), :]
        sup_i = jnp.max(jnp.where(idx == i, supp, 0.0))
        alive = 1.0 - sup_i
        cand = jnp.where((row > _NMS_THRESH) & (idx > i), alive, 0.0)
        return jnp.maximum(supp, cand)

    lvl = pl.program_id(1)
    trip = jnp.where(lvl == 3, 768, jnp.where(lvl == 4, 192, 1000))

    def body4(i4, supp):
        i = 4 * i4
        supp = body(i, supp)
        supp = body(i + 1, supp)
        supp = body(i + 2, supp)
        return body(i + 3, supp)

    supp0 = jnp.zeros((1, _P), dtype=jnp.float32)
    supp = lax.fori_loop(0, trip // 4, body4, supp0)
    keep = valid & (supp[0] == 0.0)
    ks_ref[0, 0, 0, :] = jnp.where(keep, sig, -jnp.inf)
    ox1_ref[0, 0, 0, :] = x1
    oy1_ref[0, 0, 0, :] = y1
    ox2_ref[0, 0, 0, :] = x2
    oy2_ref[0, 0, 0, :] = y2


def _run_nms(scores_p, x1, y1, x2, y2):
    ni, nl, _ = scores_p.shape
    r = lambda a: a.reshape(ni, nl, 1, _P)
    spec = pl.BlockSpec((1, 1, 1, _P), lambda i, l: (i, l, 0, 0))
    shp = jax.ShapeDtypeStruct((ni, nl, 1, _P), jnp.float32)
    outs = pl.pallas_call(
        _nms_kernel,
        grid=(ni, nl),
        in_specs=[spec] * 5,
        out_specs=[spec] * 5,
        out_shape=[shp] * 5,
        scratch_shapes=[pltpu.VMEM((_P, _P), jnp.float32)],
    )(r(scores_p), r(x1), r(y1), r(x2), r(y2))
    return [o.reshape(ni, nl, _P) for o in outs]


def kernel(proposals, objectness, num_anchors_per_level):
    ni = objectness.shape[0]
    obj = objectness.reshape(ni, -1).astype(jnp.float32)
    props = proposals.astype(jnp.float32)
    off = 0
    sc, bx = [], []
    for n in _NAPL:
        k = min(_PRE_NMS_TOP_N, n)
        v, idx = lax.top_k(obj[:, off:off + n], k)
        b = jnp.take_along_axis(props[:, off:off + n, :], idx[:, :, None],
                                axis=1)
        pad = _P - k
        v = jnp.pad(v, ((0, 0), (0, pad)), constant_values=-jnp.inf)
        b = jnp.pad(b, ((0, 0), (0, pad), (0, 0)))
        sc.append(v)
        bx.append(b)
        off += n
    scores_p = jnp.stack(sc, 1)
    boxes_p = jnp.stack(bx, 1)
    ks, ox1, oy1, ox2, oy2 = _run_nms(
        scores_p, boxes_p[..., 0], boxes_p[..., 1],
        boxes_p[..., 2], boxes_p[..., 3])
    flat_ks = ks.reshape(ni, -1)
    topv, topi = lax.top_k(flat_ks, _POST_NMS_TOP_N)
    fb = jnp.stack([ox1, oy1, ox2, oy2], -1).reshape(ni, -1, 4)
    selb = jnp.take_along_axis(fb, topi[:, :, None], axis=1)
    finv = topv > -jnp.inf
    boxes_out = jnp.where(finv[:, :, None], selb, 0.0)
    scores_out = jnp.where(finv, topv, 0.0)
    return boxes_out, scores_out
